# Initial kernel scaffold; baseline (speedup 1.0000x reference)
import jax
import jax.numpy as jnp
from jax.experimental import pallas as pl


def kernel(x, single_w, pair_w):
    raise NotImplementedError("write your pallas kernel here")


# speedup vs baseline: 6.6691x; 6.6691x over previous
import jax
import jax.numpy as jnp
from jax.experimental import pallas as pl
from jax.experimental.pallas import tpu as pltpu

L = 128
B = 64
NV = 20
PJ = NV * NV
LAM_PAIR = 0.2 * (L - 1)


def _mrf_kernel(x_ref, sw_ref, pw_ref, out_ref, obig_ref, racc_ref, accv_ref):
    i = pl.program_id(0)
    pwb = pw_ref[0]

    @pl.when(i == 0)
    def _init():
        xf = x_ref[...].astype(jnp.float32)
        q = jax.lax.broadcasted_iota(jnp.int32, (L, L * NV), 1)
        lrow = jax.lax.broadcasted_iota(jnp.int32, (L, L * NV), 0)
        rep = (q // NV == lrow).astype(jnp.float32)
        xrep = jax.lax.dot_general(
            xf, rep, (((1,), (0,)), ((), ())),
            precision=jax.lax.Precision.HIGHEST,
            preferred_element_type=jnp.float32)
        kq = (jax.lax.broadcasted_iota(jnp.int32, (B, L * NV), 1) % NV
              ).astype(jnp.float32)
        obig_ref[0:B, :] = (xrep == kq).astype(jnp.float32)
        accv_ref[...] = jnp.zeros_like(accv_ref)
        racc_ref[...] = jnp.zeros_like(racc_ref)

    racc_ref[...] += pwb * pwb + jnp.abs(pwb)

    lane = jax.lax.broadcasted_iota(jnp.int32, (1, L * NV), 1)
    l_idx = lane // NV
    k_idx = lane - l_idx * NV
    o2 = obig_ref[0:B, :]
    e19 = (k_idx == NV - 1).astype(jnp.float32)
    v = jnp.where(l_idx > i, o2, 0.0) + jnp.where(l_idx < i, e19, 0.0)
    obig_ref[B:2 * B, :] = v

    w = pwb.reshape(L, NV, NV).swapaxes(1, 2).reshape(L * NV, NV)
    r = jax.lax.dot_general(
        obig_ref[...], w, (((1,), (0,)), ((), ())),
        precision=jax.lax.Precision.HIGHEST,
        preferred_element_type=jnp.float32)
    s_full = r[0:B, :]
    ac = r[B:2 * B, :]

    rowi = pw_ref[0, pl.ds(i, 1), :]
    plane = jax.lax.broadcasted_iota(jnp.int32, (1, PJ), 1)
    pj = plane // NV
    pk = plane - pj * NV
    d400 = jnp.where(pj == pk, rowi, 0.0)
    segr = jax.lax.broadcasted_iota(jnp.int32, (PJ, NV), 0) // NV
    segc = jax.lax.broadcasted_iota(jnp.int32, (PJ, NV), 1)
    seg = (segr == segc).astype(jnp.float32)
    d = jax.lax.dot_general(
        d400, seg, (((1,), (0,)), ((), ())),
        precision=jax.lax.Precision.HIGHEST,
        preferred_element_type=jnp.float32)

    swrow = sw_ref[pl.ds(i, 1), :]

    logits = swrow + d + ac
    m = jnp.max(logits, axis=1, keepdims=True)
    te = jnp.log(jnp.sum(jnp.exp(logits - m), axis=1, keepdims=True)) + m

    xf_all = x_ref[...].astype(jnp.float32)
    col_mask = (jax.lax.broadcasted_iota(jnp.int32, (1, L), 1) == i)
    xcol = jnp.sum(jnp.where(col_mask, xf_all, 0.0), axis=1, keepdims=True)
    kro = jax.lax.broadcasted_iota(jnp.int32, (B, NV), 1).astype(jnp.float32)
    oi = (xcol == kro).astype(jnp.float32)
    sp = jnp.sum(oi * (swrow + s_full), axis=1, keepdims=True)

    accv_ref[...] += te - sp

    @pl.when(i == L - 1)
    def _fin():
        sw = sw_ref[...]
        reg_s = jnp.sum(sw * sw + jnp.abs(sw), keepdims=True)
        fe = jnp.sum(accv_ref[...], keepdims=True) / B
        rp = jnp.sum(racc_ref[...], keepdims=True)
        out_ref[...] = fe + reg_s + LAM_PAIR * rp


def _run(x, sw2, pw3):
    return pl.pallas_call(
        _mrf_kernel,
        grid=(L,),
        in_specs=[
            pl.BlockSpec((B, L), lambda i: (0, 0)),
            pl.BlockSpec((L, NV), lambda i: (0, 0)),
            pl.BlockSpec((1, L, PJ), lambda i: (i, 0, 0)),
        ],
        out_specs=pl.BlockSpec((1, 1), lambda i: (0, 0)),
        out_shape=jax.ShapeDtypeStruct((1, 1), jnp.float32),
        scratch_shapes=[
            pltpu.VMEM((2 * B, L * NV), jnp.float32),
            pltpu.VMEM((L, PJ), jnp.float32),
            pltpu.VMEM((B, 1), jnp.float32),
        ],
    )(x, sw2, pw3)


def kernel(x, single_w, pair_w):
    sw2 = single_w.reshape(L, NV)
    pw3 = pair_w.reshape(L, L, PJ)
    return _run(x, sw2, pw3)[0, 0]


# speedup vs baseline: 7.9624x; 1.1939x over previous
import jax
import jax.numpy as jnp
from jax.experimental import pallas as pl
from jax.experimental.pallas import tpu as pltpu

L = 128
B = 64
NV = 20
PJ = NV * NV
LAM_PAIR = 0.2 * (L - 1)


def _mrf_kernel(x_ref, sw_ref, pw_ref, out_ref, obig_ref, racc_ref, accv_ref):
    i = pl.program_id(0)
    pwb = pw_ref[0]

    @pl.when(i == 0)
    def _init():
        xf = x_ref[...].astype(jnp.float32)
        q = jax.lax.broadcasted_iota(jnp.int32, (L, L * NV), 1)
        lrow = jax.lax.broadcasted_iota(jnp.int32, (L, L * NV), 0)
        rep = (q // NV == lrow).astype(jnp.float32)
        xrep = jax.lax.dot_general(
            xf, rep, (((1,), (0,)), ((), ())),
            precision=jax.lax.Precision.DEFAULT,
            preferred_element_type=jnp.float32)
        kq = (jax.lax.broadcasted_iota(jnp.int32, (B, L * NV), 1) % NV
              ).astype(jnp.float32)
        obig_ref[0:B, :] = (xrep == kq).astype(jnp.float32)
        accv_ref[...] = jnp.zeros_like(accv_ref)
        racc_ref[...] = jnp.zeros_like(racc_ref)

    racc_ref[...] += pwb * pwb + jnp.abs(pwb)

    lane = jax.lax.broadcasted_iota(jnp.int32, (1, L * NV), 1)
    l_idx = lane // NV
    k_idx = lane - l_idx * NV
    o2 = obig_ref[0:B, :]
    e19 = (k_idx == NV - 1).astype(jnp.float32)
    v = jnp.where(l_idx > i, o2, 0.0) + jnp.where(l_idx < i, e19, 0.0)
    obig_ref[B:2 * B, :] = v

    w = pwb.reshape(L, NV, NV).swapaxes(1, 2).reshape(L * NV, NV)
    r = jax.lax.dot_general(
        obig_ref[...], w, (((1,), (0,)), ((), ())),
        precision=jax.lax.Precision.DEFAULT,
        preferred_element_type=jnp.float32)
    s_full = r[0:B, :]
    ac = r[B:2 * B, :]

    rowi = pw_ref[0, pl.ds(i, 1), :]
    plane = jax.lax.broadcasted_iota(jnp.int32, (1, PJ), 1)
    pj = plane // NV
    pk = plane - pj * NV
    d400 = jnp.where(pj == pk, rowi, 0.0)
    segr = jax.lax.broadcasted_iota(jnp.int32, (PJ, NV), 0) // NV
    segc = jax.lax.broadcasted_iota(jnp.int32, (PJ, NV), 1)
    seg = (segr == segc).astype(jnp.float32)
    d = jax.lax.dot_general(
        d400, seg, (((1,), (0,)), ((), ())),
        precision=jax.lax.Precision.DEFAULT,
        preferred_element_type=jnp.float32)

    swrow = sw_ref[pl.ds(i, 1), :]

    logits = swrow + d + ac
    m = jnp.max(logits, axis=1, keepdims=True)
    te = jnp.log(jnp.sum(jnp.exp(logits - m), axis=1, keepdims=True)) + m

    xf_all = x_ref[...].astype(jnp.float32)
    col_mask = (jax.lax.broadcasted_iota(jnp.int32, (1, L), 1) == i)
    xcol = jnp.sum(jnp.where(col_mask, xf_all, 0.0), axis=1, keepdims=True)
    kro = jax.lax.broadcasted_iota(jnp.int32, (B, NV), 1).astype(jnp.float32)
    oi = (xcol == kro).astype(jnp.float32)
    sp = jnp.sum(oi * (swrow + s_full), axis=1, keepdims=True)

    accv_ref[...] += te - sp

    @pl.when(i == L - 1)
    def _fin():
        sw = sw_ref[...]
        reg_s = jnp.sum(sw * sw + jnp.abs(sw), keepdims=True)
        fe = jnp.sum(accv_ref[...], keepdims=True) / B
        rp = jnp.sum(racc_ref[...], keepdims=True)
        out_ref[...] = fe + reg_s + LAM_PAIR * rp


def _run(x, sw2, pw3):
    return pl.pallas_call(
        _mrf_kernel,
        grid=(L,),
        in_specs=[
            pl.BlockSpec((B, L), lambda i: (0, 0)),
            pl.BlockSpec((L, NV), lambda i: (0, 0)),
            pl.BlockSpec((1, L, PJ), lambda i: (i, 0, 0)),
        ],
        out_specs=pl.BlockSpec((1, 1), lambda i: (0, 0)),
        out_shape=jax.ShapeDtypeStruct((1, 1), jnp.float32),
        scratch_shapes=[
            pltpu.VMEM((2 * B, L * NV), jnp.float32),
            pltpu.VMEM((L, PJ), jnp.float32),
            pltpu.VMEM((B, 1), jnp.float32),
        ],
    )(x, sw2, pw3)


def kernel(x, single_w, pair_w):
    sw2 = single_w.reshape(L, NV)
    pw3 = pair_w.reshape(L, L, PJ)
    return _run(x, sw2, pw3)[0, 0]


# speedup vs baseline: 7.9672x; 1.0006x over previous
import jax
import jax.numpy as jnp
from jax.experimental import pallas as pl
from jax.experimental.pallas import tpu as pltpu

L = 128
B = 64
NV = 20
PJ = NV * NV
LAM_PAIR = 0.2 * (L - 1)


def _mrf_kernel(x_ref, sw_ref, pw_ref, out_ref, obig_ref, racc_ref, accv_ref):
    i = pl.program_id(0)
    pwb = pw_ref[...]

    @pl.when(i == 0)
    def _init():
        xf = x_ref[...].astype(jnp.float32)
        q = jax.lax.broadcasted_iota(jnp.int32, (L, L * NV), 1)
        lrow = jax.lax.broadcasted_iota(jnp.int32, (L, L * NV), 0)
        rep = (q // NV == lrow).astype(jnp.float32)
        xrep = jax.lax.dot_general(
            xf, rep, (((1,), (0,)), ((), ())),
            precision=jax.lax.Precision.DEFAULT,
            preferred_element_type=jnp.float32)
        kq = (jax.lax.broadcasted_iota(jnp.int32, (B, L * NV), 1) % NV
              ).astype(jnp.float32)
        obig_ref[0:B, :] = (xrep == kq).astype(jnp.float32)
        accv_ref[...] = jnp.zeros_like(accv_ref)
        racc_ref[...] = jnp.zeros_like(racc_ref)

    racc_ref[...] += pwb * pwb + jnp.abs(pwb)

    lane = jax.lax.broadcasted_iota(jnp.int32, (1, L * NV), 1)
    l_idx = lane // NV
    k_idx = lane - l_idx * NV
    o2 = obig_ref[0:B, :]
    e19 = (k_idx == NV - 1).astype(jnp.float32)
    v = jnp.where(l_idx > i, o2, 0.0) + jnp.where(l_idx < i, e19, 0.0)
    obig_ref[B:2 * B, :] = v

    w = pwb.reshape(L, NV, NV).swapaxes(1, 2).reshape(L * NV, NV)
    r = jax.lax.dot_general(
        obig_ref[...], w, (((1,), (0,)), ((), ())),
        precision=jax.lax.Precision.DEFAULT,
        preferred_element_type=jnp.float32)
    s_full = r[0:B, :]
    ac = r[B:2 * B, :]

    rowi = pw_ref[pl.ds(i, 1), :]
    plane = jax.lax.broadcasted_iota(jnp.int32, (1, PJ), 1)
    pj = plane // NV
    pk = plane - pj * NV
    d400 = jnp.where(pj == pk, rowi, 0.0)
    segr = jax.lax.broadcasted_iota(jnp.int32, (PJ, NV), 0) // NV
    segc = jax.lax.broadcasted_iota(jnp.int32, (PJ, NV), 1)
    seg = (segr == segc).astype(jnp.float32)
    d = jax.lax.dot_general(
        d400, seg, (((1,), (0,)), ((), ())),
        precision=jax.lax.Precision.DEFAULT,
        preferred_element_type=jnp.float32)

    swrow = sw_ref[pl.ds(i, 1), :]

    logits = swrow + d + ac
    m = jnp.max(logits, axis=1, keepdims=True)
    te = jnp.log(jnp.sum(jnp.exp(logits - m), axis=1, keepdims=True)) + m

    xf_all = x_ref[...].astype(jnp.float32)
    col_mask = (jax.lax.broadcasted_iota(jnp.int32, (1, L), 1) == i)
    xcol = jnp.sum(jnp.where(col_mask, xf_all, 0.0), axis=1, keepdims=True)
    kro = jax.lax.broadcasted_iota(jnp.int32, (B, NV), 1).astype(jnp.float32)
    oi = (xcol == kro).astype(jnp.float32)
    sp = jnp.sum(oi * (swrow + s_full), axis=1, keepdims=True)

    accv_ref[...] += te - sp

    @pl.when(i == L - 1)
    def _fin():
        sw = sw_ref[...]
        reg_s = jnp.sum(sw * sw + jnp.abs(sw), keepdims=True)
        fe = jnp.sum(accv_ref[...], keepdims=True) / B
        rp = jnp.sum(racc_ref[...], keepdims=True)
        out_ref[...] = fe + reg_s + LAM_PAIR * rp


def _run(x, sw2, pw3):
    return pl.pallas_call(
        _mrf_kernel,
        grid=(L,),
        in_specs=[
            pl.BlockSpec((B, L), lambda i: (0, 0)),
            pl.BlockSpec((L, NV), lambda i: (0, 0)),
            pl.BlockSpec((L, PJ), lambda i: (i, 0)),
        ],
        out_specs=pl.BlockSpec((1, 1), lambda i: (0, 0)),
        out_shape=jax.ShapeDtypeStruct((1, 1), jnp.float32),
        scratch_shapes=[
            pltpu.VMEM((2 * B, L * NV), jnp.float32),
            pltpu.VMEM((L, PJ), jnp.float32),
            pltpu.VMEM((B, 1), jnp.float32),
        ],
    )(x, sw2, pw3)


def kernel(x, single_w, pair_w):
    sw2 = single_w.reshape(L, NV)
    pw2 = pair_w.reshape(L * L, PJ)
    return _run(x, sw2, pw2)[0, 0]


# speedup vs baseline: 25.7839x; 3.2362x over previous
import jax
import jax.numpy as jnp
from jax.experimental import pallas as pl
from jax.experimental.pallas import tpu as pltpu

L = 128
B = 64
NV = 20
PJ = NV * NV
NI = 8
NSTEP = L // NI
LAM_PAIR = 0.2 * (L - 1)


def _mrf_kernel(x_ref, sw_ref, pw_ref, out_ref, obig_ref, racc_ref, accv_ref):
    step = pl.program_id(0)
    raw = pw_ref[...]

    @pl.when(step == 0)
    def _init():
        xf = x_ref[...].astype(jnp.float32)
        q = jax.lax.broadcasted_iota(jnp.int32, (L, L * NV), 1)
        lrow = jax.lax.broadcasted_iota(jnp.int32, (L, L * NV), 0)
        rep = (q // NV == lrow).astype(jnp.float32)
        xrep = jax.lax.dot_general(
            xf, rep, (((1,), (0,)), ((), ())),
            preferred_element_type=jnp.float32)
        kq = (jax.lax.broadcasted_iota(jnp.int32, (B, L * NV), 1) % NV
              ).astype(jnp.float32)
        obig_ref[0:B, :] = (xrep == kq).astype(jnp.float32)
        accv_ref[...] = jnp.zeros_like(accv_ref)
        racc_ref[...] = jnp.zeros_like(racc_ref)

    racc_ref[...] += raw * raw + jnp.abs(raw)

    r4 = raw.reshape(NI, 16, 25, 128)
    parts = [r4[:, :, rho, :].reshape(NI * 16, 128) for rho in range(25)]
    sl_all = jnp.concatenate(parts, axis=1)
    sl3 = sl_all.reshape(128, 8, PJ)

    lane = jax.lax.broadcasted_iota(jnp.int32, (1, L * NV), 1)
    l_idx = lane // NV
    k_idx = lane - l_idx * NV
    e19 = (k_idx == NV - 1).astype(jnp.float32)
    o2 = obig_ref[0:B, :]

    xf_all = x_ref[...].astype(jnp.float32)
    kro = jax.lax.broadcasted_iota(jnp.int32, (B, NV), 1).astype(jnp.float32)

    plane = jax.lax.broadcasted_iota(jnp.int32, (1, PJ), 1)
    pj_id = plane // NV
    pk_id = plane - pj_id * NV
    segr = jax.lax.broadcasted_iota(jnp.int32, (PJ, NV), 0) // NV
    segc = jax.lax.broadcasted_iota(jnp.int32, (PJ, NV), 1)
    seg = (segr == segc).astype(jnp.float32)
    c128 = jax.lax.broadcasted_iota(jnp.int32, (128, NV), 0)
    v20 = jax.lax.broadcasted_iota(jnp.int32, (128, NV), 1)
    swr = jax.lax.broadcasted_iota(jnp.int32, (NV, 128), 0)
    swc = jax.lax.broadcasted_iota(jnp.int32, (NV, 128), 1)
    sw_all = sw_ref[...]

    acc_new = jnp.zeros((B, 1), jnp.float32)

    for s in range(NI):
        i = step * NI + s
        pwb = jax.lax.slice(
            sl3, (s * 16, 0, 0), ((s + 1) * 16, 8, PJ)).reshape(L, PJ)

        v = jnp.where(l_idx > i, o2, 0.0) + jnp.where(l_idx < i, e19, 0.0)
        obig_ref[B:2 * B, :] = v

        w = pwb.reshape(L, NV, NV).swapaxes(1, 2).reshape(L * NV, NV)
        r = jax.lax.dot_general(
            obig_ref[...], w, (((1,), (0,)), ((), ())),
            preferred_element_type=jnp.float32)
        s_full = r[0:B, :]
        ac = r[B:2 * B, :]

        rmask = (jax.lax.broadcasted_iota(jnp.int32, (L, 1), 0) == i)
        rowi = jnp.sum(jnp.where(rmask, pwb, 0.0), axis=0, keepdims=True)
        d400 = jnp.where(pj_id == pk_id, rowi, 0.0)
        d = jax.lax.dot_general(
            d400, seg, (((1,), (0,)), ((), ())),
            preferred_element_type=jnp.float32)

        swval = 128 * swr + swc - NV * i
        swmask = (swval >= 0) & (swval < NV)
        swcol = jnp.sum(jnp.where(swmask, sw_all, 0.0), axis=0,
                        keepdims=True)
        pperm = (c128 == (NV * i + v20) % 128).astype(jnp.float32)
        swrow = jax.lax.dot_general(
            swcol, pperm, (((1,), (0,)), ((), ())),
            preferred_element_type=jnp.float32)

        logits = swrow + d + ac
        m = jnp.max(logits, axis=1, keepdims=True)
        te = jnp.log(jnp.sum(jnp.exp(logits - m), axis=1, keepdims=True)) + m

        col_mask = (jax.lax.broadcasted_iota(jnp.int32, (1, L), 1) == i)
        xcol = jnp.sum(jnp.where(col_mask, xf_all, 0.0), axis=1, keepdims=True)
        oi = (xcol == kro).astype(jnp.float32)
        sp = jnp.sum(oi * (swrow + s_full), axis=1, keepdims=True)

        acc_new = acc_new + te - sp

    accv_ref[...] += acc_new

    @pl.when(step == NSTEP - 1)
    def _fin():
        sw = sw_ref[...]
        reg_s = jnp.sum(sw * sw + jnp.abs(sw), keepdims=True)
        fe = jnp.sum(accv_ref[...], keepdims=True) / B
        rp = jnp.sum(racc_ref[...], keepdims=True)
        out_ref[...] = fe + reg_s + LAM_PAIR * rp


def _run(x, sw2, pw2):
    return pl.pallas_call(
        _mrf_kernel,
        grid=(NSTEP,),
        in_specs=[
            pl.BlockSpec((B, L), lambda i: (0, 0)),
            pl.BlockSpec((NV, 128), lambda i: (0, 0)),
            pl.BlockSpec((NI * PJ, 128), lambda i: (i, 0)),
        ],
        out_specs=pl.BlockSpec((1, 1), lambda i: (0, 0)),
        out_shape=jax.ShapeDtypeStruct((1, 1), jnp.float32),
        scratch_shapes=[
            pltpu.VMEM((2 * B, L * NV), jnp.float32),
            pltpu.VMEM((NI * PJ, 128), jnp.float32),
            pltpu.VMEM((B, 1), jnp.float32),
        ],
    )(x, sw2, pw2)


def kernel(x, single_w, pair_w):
    sw2 = single_w.reshape(NV, 128)
    pw2 = pair_w.reshape(L * PJ, 128)
    return _run(x, sw2, pw2)[0, 0]
